# baseline (speedup 1.0000x reference)
import functools

import jax
import jax.numpy as jnp
from jax import lax
from jax.experimental import pallas as pl
from jax.experimental.pallas import tpu as pltpu
from jax.experimental.pallas import tpu_sc as plsc

_NUM_SEG = 512
_NC = 1
_NS = 16
_NW = _NC * _NS
_LANES = 16
_TAB_W = 528
_BIG = 2**30


def _sc_starts_body(n, chunk, idx_hbm, out_hbm, idx_v, tab_v, sem):
    wid = lax.axis_index("s") * _NC + lax.axis_index("c")
    base = wid * chunk
    last_len = n - (_NW - 1) * chunk
    is_tail = wid == _NW - 1

    @pl.when(wid == 0)
    def _():
        idx_v[pl.ds(0, _LANES)] = jnp.full((_LANES,), -1, jnp.int32)
        pltpu.async_copy(
            idx_hbm.at[pl.ds(0, chunk)], idx_v.at[pl.ds(_LANES, chunk)], sem
        )

    @pl.when(jnp.logical_and(wid > 0, jnp.logical_not(is_tail)))
    def _():
        pltpu.async_copy(
            idx_hbm.at[pl.ds(base - _LANES, chunk + _LANES)],
            idx_v.at[pl.ds(0, chunk + _LANES)],
            sem,
        )

    @pl.when(is_tail)
    def _():
        pltpu.async_copy(
            idx_hbm.at[pl.ds(base - _LANES, last_len + _LANES)],
            idx_v.at[pl.ds(0, last_len + _LANES)],
            sem,
        )

    zeros = jnp.zeros((_LANES,), jnp.int32)

    @plsc.parallel_loop(0, _TAB_W // _LANES, 1, unroll=4)
    def _(i):
        tab_v[pl.ds(i * _LANES, _LANES)] = zeros

    @pl.when(wid == 0)
    def _():
        pltpu.make_async_copy(
            idx_hbm.at[pl.ds(0, chunk)], idx_v.at[pl.ds(_LANES, chunk)], sem
        ).wait()

    @pl.when(jnp.logical_and(wid > 0, jnp.logical_not(is_tail)))
    def _():
        pltpu.make_async_copy(
            idx_hbm.at[pl.ds(base - _LANES, chunk + _LANES)],
            idx_v.at[pl.ds(0, chunk + _LANES)],
            sem,
        ).wait()

    @pl.when(is_tail)
    def _():
        pltpu.make_async_copy(
            idx_hbm.at[pl.ds(base - _LANES, last_len + _LANES)],
            idx_v.at[pl.ds(0, last_len + _LANES)],
            sem,
        ).wait()
        fill = jnp.full((_LANES,), _NUM_SEG, jnp.int32)

        @plsc.parallel_loop(0, (chunk - last_len) // _LANES, 1, unroll=4)
        def _(j):
            idx_v[pl.ds(_LANES + last_len + j * _LANES, _LANES)] = fill

    lane = lax.iota(jnp.int32, _LANES)

    @plsc.parallel_loop(0, chunk // _LANES, 1, unroll=4)
    def _(i):
        v = idx_v[pl.ds(_LANES + i * _LANES, _LANES)]
        p = idx_v[pl.ds(_LANES - 1 + i * _LANES, _LANES)]
        pos1 = (base + 1 + i * _LANES) + lane
        plsc.store_scatter(tab_v, [v], pos1, mask=v != p)

    pltpu.sync_copy(tab_v, out_hbm.at[wid])


@functools.lru_cache(maxsize=None)
def _make_sc_starts(n):
    chunk = -(-n // _NW)
    chunk += (-chunk) % _LANES
    mesh = plsc.VectorSubcoreMesh(
        core_axis_name="c", subcore_axis_name="s", num_cores=_NC
    )
    return pl.kernel(
        functools.partial(_sc_starts_body, n, chunk),
        mesh=mesh,
        out_type=jax.ShapeDtypeStruct((_NW, _TAB_W), jnp.int32),
        scratch_types=[
            pltpu.VMEM((chunk + _LANES,), jnp.int32),
            pltpu.VMEM((_TAB_W,), jnp.int32),
            pltpu.SemaphoreType.DMA,
        ],
        compiler_params=pltpu.CompilerParams(needs_layout_passes=False),
    )


def _tc_mlp_body(n, parts_ref, w1_ref, b1_ref, w2_ref, b2_ref, out_ref):
    tab = jnp.sum(parts_ref[...], axis=0)
    lanes = lax.iota(jnp.int32, 1024)
    a = jnp.concatenate([tab, jnp.zeros((1024 - _TAB_W,), jnp.int32)])
    a = jnp.where(jnp.logical_and(lanes < _NUM_SEG, a > 0), a - 1, _BIG)
    a = jnp.where(lanes == _NUM_SEG, n, a)
    k = 1
    while k <= _NUM_SEG:
        shifted = jnp.concatenate([a[k:], jnp.full((k,), _BIG, jnp.int32)])
        a = jnp.minimum(a, shifted)
        k *= 2
    nxt = jnp.concatenate([a[1:], jnp.full((1,), _BIG, jnp.int32)])
    counts = (nxt - a)[:_NUM_SEG].astype(jnp.float32)
    inv = (1.0 / counts).reshape(_NUM_SEG, 1)
    h = jnp.maximum(inv * w1_ref[...] + b1_ref[...], 0.0)
    out_ref[...] = (
        jnp.dot(h, w2_ref[...], preferred_element_type=jnp.float32)
        + b2_ref[...]
    )


def kernel(x, batch_index, W1, b1, W2, b2):
    del x
    idx = batch_index.astype(jnp.int32)
    n_orig = n = idx.shape[0]
    if n % _LANES != 0:
        pad = (-n) % _LANES
        idx = jnp.concatenate([idx, jnp.full((pad,), _NUM_SEG, jnp.int32)])
        n += pad
    parts = _make_sc_starts(n)(idx)
    out = pl.pallas_call(
        functools.partial(_tc_mlp_body, n_orig),
        out_shape=jax.ShapeDtypeStruct((_NUM_SEG, 128), jnp.float32),
    )(parts, W1, b1.reshape(1, 128), W2, b2.reshape(1, 128))
    return out


# speedup vs baseline: 1.0175x; 1.0175x over previous
import functools

import jax
import jax.numpy as jnp
from jax import lax
from jax.experimental import pallas as pl
from jax.experimental.pallas import tpu as pltpu
from jax.experimental.pallas import tpu_sc as plsc

_NUM_SEG = 512
_NC = 1
_NS = 16
_NW = _NC * _NS
_LANES = 16
_HALF = 528
_TAB_W = 2 * _HALF


def _sc_bounds_body(n, chunk, idx_hbm, out_hbm, idx_v, tab_v, sem):
    wid = lax.axis_index("s") * _NC + lax.axis_index("c")
    base = wid * chunk
    last_len = n - (_NW - 1) * chunk
    is_tail = wid == _NW - 1

    @pl.when(wid == 0)
    def _():
        idx_v[pl.ds(0, _LANES)] = jnp.full((_LANES,), -1, jnp.int32)
        pltpu.async_copy(
            idx_hbm.at[pl.ds(0, chunk + _LANES)],
            idx_v.at[pl.ds(_LANES, chunk + _LANES)],
            sem,
        )

    @pl.when(jnp.logical_and(wid > 0, jnp.logical_not(is_tail)))
    def _():
        pltpu.async_copy(
            idx_hbm.at[pl.ds(base - _LANES, chunk + 2 * _LANES)],
            idx_v.at[pl.ds(0, chunk + 2 * _LANES)],
            sem,
        )

    @pl.when(is_tail)
    def _():
        pltpu.async_copy(
            idx_hbm.at[pl.ds(base - _LANES, last_len + _LANES)],
            idx_v.at[pl.ds(0, last_len + _LANES)],
            sem,
        )

    zeros = jnp.zeros((_LANES,), jnp.int32)

    @plsc.parallel_loop(0, _TAB_W // _LANES, 1, unroll=4)
    def _(i):
        tab_v[pl.ds(i * _LANES, _LANES)] = zeros

    @pl.when(wid == 0)
    def _():
        pltpu.make_async_copy(
            idx_hbm.at[pl.ds(0, chunk + _LANES)],
            idx_v.at[pl.ds(_LANES, chunk + _LANES)],
            sem,
        ).wait()

    @pl.when(jnp.logical_and(wid > 0, jnp.logical_not(is_tail)))
    def _():
        pltpu.make_async_copy(
            idx_hbm.at[pl.ds(base - _LANES, chunk + 2 * _LANES)],
            idx_v.at[pl.ds(0, chunk + 2 * _LANES)],
            sem,
        ).wait()

    @pl.when(is_tail)
    def _():
        pltpu.make_async_copy(
            idx_hbm.at[pl.ds(base - _LANES, last_len + _LANES)],
            idx_v.at[pl.ds(0, last_len + _LANES)],
            sem,
        ).wait()
        fill = jnp.full((_LANES,), _NUM_SEG, jnp.int32)

        @plsc.parallel_loop(0, (chunk - last_len) // _LANES + 1, 1, unroll=4)
        def _(j):
            idx_v[pl.ds(_LANES + last_len + j * _LANES, _LANES)] = fill

    lane = lax.iota(jnp.int32, _LANES)
    half = jnp.full((_LANES,), _HALF, jnp.int32)

    @plsc.parallel_loop(0, chunk // _LANES, 1, unroll=4)
    def _(i):
        v = idx_v[pl.ds(_LANES + i * _LANES, _LANES)]
        p = idx_v[pl.ds(_LANES - 1 + i * _LANES, _LANES)]
        q = idx_v[pl.ds(_LANES + 1 + i * _LANES, _LANES)]
        pos = (base + i * _LANES) + lane
        plsc.store_scatter(tab_v, [v], pos, mask=v != p)
        plsc.store_scatter(tab_v, [v + half], pos + 1, mask=v != q)
    pltpu.sync_copy(tab_v, out_hbm.at[wid])


@functools.lru_cache(maxsize=None)
def _make_sc_bounds(n):
    chunk = -(-n // _NW)
    chunk += (-chunk) % _LANES
    mesh = plsc.VectorSubcoreMesh(
        core_axis_name="c", subcore_axis_name="s", num_cores=_NC
    )
    return pl.kernel(
        functools.partial(_sc_bounds_body, n, chunk),
        mesh=mesh,
        out_type=jax.ShapeDtypeStruct((_NW, _TAB_W), jnp.int32),
        scratch_types=[
            pltpu.VMEM((chunk + 2 * _LANES,), jnp.int32),
            pltpu.VMEM((_TAB_W,), jnp.int32),
            pltpu.SemaphoreType.DMA,
        ],
        compiler_params=pltpu.CompilerParams(needs_layout_passes=False),
    )


def _tc_mlp_body(parts_ref, w1_ref, b1_ref, w2_ref, b2_ref, out_ref):
    tab = jnp.sum(parts_ref[...], axis=0)
    counts = (tab[_HALF : _HALF + _NUM_SEG] - tab[:_NUM_SEG]).astype(
        jnp.float32
    )
    inv = (1.0 / counts).reshape(_NUM_SEG, 1)
    h = jnp.maximum(inv * w1_ref[...] + b1_ref[...], 0.0)
    out_ref[...] = (
        jnp.dot(h, w2_ref[...], preferred_element_type=jnp.float32)
        + b2_ref[...]
    )


def kernel(x, batch_index, W1, b1, W2, b2):
    del x
    idx = batch_index.astype(jnp.int32)
    n = idx.shape[0]
    if n % _LANES != 0:
        pad = (-n) % _LANES
        idx = jnp.concatenate([idx, jnp.full((pad,), _NUM_SEG, jnp.int32)])
        n += pad
    parts = _make_sc_bounds(n)(idx)
    out = pl.pallas_call(
        _tc_mlp_body,
        out_shape=jax.ShapeDtypeStruct((_NUM_SEG, 128), jnp.float32),
    )(parts, W1, b1.reshape(1, 128), W2, b2.reshape(1, 128))
    return out
